# baseline (speedup 1.0000x reference)
import functools

import jax
import jax.numpy as jnp
from jax import lax
from jax.experimental import pallas as pl
from jax.experimental.pallas import tpu as pltpu
from jax.experimental.pallas import tpu_sc as plsc

NC = 2
NS = 16
NW = NC * NS
L = 16

B = 16384
BPW = B // NW
CH = 128
NCH = BPW // CH

_mesh = plsc.VectorSubcoreMesh(core_axis_name="c", subcore_axis_name="s")


@functools.partial(
    pl.kernel,
    mesh=_mesh,
    compiler_params=pltpu.CompilerParams(needs_layout_passes=False),
    out_type=jax.ShapeDtypeStruct((NW, NCH, CH), jnp.float32),
    scratch_types=[
        pltpu.VMEM((2 * BPW,), jnp.int32),
        pltpu.VMEM((NCH, CH), jnp.int32),
        pltpu.VMEM((NCH, CH), jnp.float32),
        pltpu.SemaphoreType.DMA,
    ],
)
def _lookup(xs_hbm, table_hbm, out_hbm, xs_v, idx_v, val_v, sem):
    wid = lax.axis_index("s") * NC + lax.axis_index("c")
    base = wid * BPW
    pltpu.sync_copy(xs_hbm.at[pl.ds(2 * base, 2 * BPW)], xs_v)
    lanes2 = lax.broadcasted_iota(jnp.int32, (L,), 0) * 2
    for j in range(NCH):
        row = idx_v.at[j]
        for i in range(CH // L):
            off = lanes2 + (2 * (j * CH + i * L))
            x0 = plsc.load_gather(xs_v, [off])
            x1 = plsc.load_gather(xs_v, [off + 1])
            row[pl.ds(i * L, L)] = x0 * 1000 + x1
    copies = [
        pltpu.async_copy(table_hbm.at[idx_v.at[j]], val_v.at[j], sem)
        for j in range(NCH)
    ]
    for c in copies:
        c.wait()
    pltpu.sync_copy(val_v, out_hbm.at[wid])


def kernel(xs, param_vec):
    return _lookup(xs.reshape(2 * B), param_vec).reshape(B)


# speedup vs baseline: 1.4961x; 1.4961x over previous
import functools

import jax
import jax.numpy as jnp
from jax import lax
from jax.experimental import pallas as pl
from jax.experimental.pallas import tpu as pltpu
from jax.experimental.pallas import tpu_sc as plsc

NC = 2
NS = 16
NW = NC * NS
L = 16

B = 16384
BPW = B // NW
CH = 128
NCH = BPW // CH

_mesh = plsc.VectorSubcoreMesh(core_axis_name="c", subcore_axis_name="s")


@functools.partial(
    pl.kernel,
    mesh=_mesh,
    out_type=jax.ShapeDtypeStruct((B,), jnp.float32),
    scratch_types=[
        pltpu.VMEM((BPW,), jnp.int32),
        pltpu.VMEM((NCH, CH), jnp.int32),
        pltpu.VMEM((BPW,), jnp.float32),
        pltpu.SemaphoreType.DMA,
    ],
)
def _lookup(packed_hbm, table_hbm, out_hbm, p_v, idx_v, val_v, sem):
    wid = lax.axis_index("s") * NC + lax.axis_index("c")
    base = wid * BPW
    pltpu.sync_copy(packed_hbm.at[pl.ds(base, BPW)], p_v)
    copies = []
    for j in range(NCH):
        row = idx_v.at[j]
        for i in range(CH // L):
            p = p_v[pl.ds(j * CH + i * L, L)]
            row[pl.ds(i * L, L)] = (p >> 16) * 1000 + (p & 0xFFFF)
        copies.append(
            pltpu.async_copy(
                table_hbm.at[row], val_v.at[pl.ds(j * CH, CH)], sem
            )
        )
    for c in copies:
        c.wait()
    pltpu.sync_copy(val_v, out_hbm.at[pl.ds(base, BPW)])


def kernel(xs, param_vec):
    packed = (xs[:, 0] << 16) | xs[:, 1]
    return _lookup(packed, param_vec)
